# baseline (speedup 1.0000x reference)
import math

import jax
import jax.numpy as jnp
from jax import lax
from jax.experimental import pallas as pl
from jax.experimental.pallas import tpu as pltpu
from jax.experimental.pallas import tpu_sc as plsc

VOCAB = 1000000
NCAT = 16
B = 4096
L = 200
K = 50

NC = 2
NS = 16
NW = NC * NS
UPW = B // NW
GROUPS = UPW // 16

IW = 128
H_IDX = UPW * L
R_IDX = UPW * K
H_STREAMS = H_IDX // IW
R_STREAMS = R_IDX // IW
H_STREAMS_PER_GROUP = 16 * L // IW
R_STREAMS_PER_GROUP = 16 * K // IW

PACKED_N = VOCAB
ROWS_A = 31256
CHUNK_A = 768
NCH_A = -(-ROWS_A // CHUNK_A)
G16_A = CHUNK_A // 16


def _pack_body_sc(table, packed_out, stage_v, pk_v):
    c = lax.axis_index("c")
    s = lax.axis_index("s")
    wid = c * NS + s

    lane = jax.lax.iota(jnp.int32, 16)
    start = wid * (VOCAB // NW)
    eff = (start // 8) * 8

    def ch_body(ch, _):
        base = jnp.minimum(ch * CHUNK_A, ROWS_A - CHUNK_A)
        pltpu.sync_copy(table.at[pl.ds(eff + base, CHUNK_A)], stage_v)

        def g_body(g, _):
            row = g * 16 + lane
            o = jnp.zeros((16,), jnp.int32)
            for cc in range(NCAT):
                col = plsc.load_gather(stage_v, [row, lane * 0 + cc])
                o = o | (col << cc)
            pk_v[pl.ds(g * 16, 16)] = o
            return 0

        lax.fori_loop(0, G16_A, g_body, 0)
        pltpu.sync_copy(pk_v.at[pl.ds(0, CHUNK_A)],
                        packed_out.at[pl.ds(eff + base, CHUNK_A)])
        return 0

    lax.fori_loop(0, NCH_A, ch_body, 0)


_pack = pl.kernel(
    _pack_body_sc,
    out_type=jax.ShapeDtypeStruct((PACKED_N,), jnp.int32),
    mesh=plsc.VectorSubcoreMesh(
        core_axis_name="c", subcore_axis_name="s", num_cores=NC, num_subcores=NS
    ),
    scratch_types=[
        pltpu.VMEM((CHUNK_A, NCAT), jnp.int32),
        pltpu.VMEM((CHUNK_A,), jnp.int32),
    ],
    compiler_params=pltpu.CompilerParams(
        use_tc_tiling_on_sc=True, needs_layout_passes=False
    ),
)


def _sc_body(packed, us_f, rec_f, p_out, q_out, idx_h, idx_r, words_h, words_r,
             psum_v, qsum_v, sem):
    c = lax.axis_index("c")
    s = lax.axis_index("s")
    wid = c * NS + s

    zero = jnp.zeros((16,), jnp.int32)
    lane = jax.lax.iota(jnp.int32, 16)

    pltpu.sync_copy(us_f.at[pl.ds(wid * H_IDX, H_IDX)], idx_h)
    pltpu.sync_copy(rec_f.at[pl.ds(wid * R_IDX, R_IDX)], idx_r)

    def fire(idx_ref, words_ref, chunk0, n):
        cps = [
            pltpu.make_async_copy(
                packed.at[idx_ref.at[pl.ds((chunk0 + j) * IW, IW)]],
                words_ref.at[pl.ds((chunk0 + j) * IW, IW)],
                sem,
            )
            for j in range(n)
        ]
        for cp in cps:
            cp.start()
        for cp in cps:
            cp.wait()

    def h_step(g, _):
        fire(idx_h, words_h, g * H_STREAMS_PER_GROUP, H_STREAMS_PER_GROUP)
        base = lane * L + g * 16 * L

        def r_body(r, accs):
            w = plsc.load_gather(words_h, [base + r])
            return tuple(accs[cc] + ((w >> cc) & 0x101) for cc in range(8))

        accs = lax.fori_loop(0, L, r_body, (zero,) * 8)
        out_base = (lane + g * 16) * NCAT
        for cc in range(8):
            plsc.store_scatter(psum_v, [out_base + cc], accs[cc] & 0xFF)
            plsc.store_scatter(psum_v, [out_base + cc + 8], (accs[cc] >> 8) & 0xFF)
        return 0

    lax.fori_loop(0, GROUPS, h_step, 0)

    def r_fire(st, _):
        fire(idx_r, words_r, st * 25, 25)
        return 0

    lax.fori_loop(0, R_STREAMS // 25, r_fire, 0)

    def r_step(g, _):
        base = lane * K + g * 16 * K

        def r_body(r, accs):
            w = plsc.load_gather(words_r, [base + r])
            return tuple(accs[cc] + ((w >> cc) & 0x101) for cc in range(8))

        accs = lax.fori_loop(0, K, r_body, (zero,) * 8)
        out_base = (lane + g * 16) * NCAT
        for cc in range(8):
            plsc.store_scatter(qsum_v, [out_base + cc], accs[cc] & 0xFF)
            plsc.store_scatter(qsum_v, [out_base + cc + 8], (accs[cc] >> 8) & 0xFF)
        return 0

    lax.fori_loop(0, GROUPS, r_step, 0)

    pltpu.sync_copy(psum_v, p_out.at[pl.ds(wid * UPW * NCAT, UPW * NCAT)])
    pltpu.sync_copy(qsum_v, q_out.at[pl.ds(wid * UPW * NCAT, UPW * NCAT)])


_sc_sums = pl.kernel(
    _sc_body,
    out_type=(
        jax.ShapeDtypeStruct((B * NCAT,), jnp.int32),
        jax.ShapeDtypeStruct((B * NCAT,), jnp.int32),
    ),
    mesh=plsc.VectorSubcoreMesh(
        core_axis_name="c", subcore_axis_name="s", num_cores=NC, num_subcores=NS
    ),
    scratch_types=[
        pltpu.VMEM((H_IDX,), jnp.int32),
        pltpu.VMEM((R_IDX,), jnp.int32),
        pltpu.VMEM((H_IDX,), jnp.int32),
        pltpu.VMEM((R_IDX,), jnp.int32),
        pltpu.VMEM((UPW * NCAT,), jnp.int32),
        pltpu.VMEM((UPW * NCAT,), jnp.int32),
        pltpu.SemaphoreType.DMA,
    ],
    compiler_params=pltpu.CompilerParams(
        use_tc_tiling_on_sc=False, needs_layout_passes=False
    ),
)


def _hell_body(p_ref, q_ref, o_ref):
    p = p_ref[...].astype(jnp.float32) * (1.0 / L)
    q = q_ref[...].astype(jnp.float32) * (1.0 / K)
    d = jnp.sqrt(p) - jnp.sqrt(q)
    o_ref[...] = jnp.sum(d * d, axis=1, keepdims=True) * (1.0 / math.sqrt(2.0))


_hell = pl.pallas_call(
    _hell_body,
    grid=(8,),
    in_specs=[
        pl.BlockSpec((B // 8, NCAT), lambda i: (i, 0)),
        pl.BlockSpec((B // 8, NCAT), lambda i: (i, 0)),
    ],
    out_specs=pl.BlockSpec((B // 8, 1), lambda i: (i, 0)),
    out_shape=jax.ShapeDtypeStruct((B, 1), jnp.float32),
)


@jax.jit
def _impl(item_categories, user_sequence, recommendations):
    packed = _pack(item_categories)
    us_f = user_sequence.reshape(B * L)
    rec_f = recommendations.reshape(B * K)
    p_sum, q_sum = _sc_sums(packed, us_f, rec_f)
    return _hell(p_sum.reshape(B, NCAT), q_sum.reshape(B, NCAT)).reshape(B)


def kernel(item_categories, user_sequence, recommendations):
    return _impl(item_categories, user_sequence, recommendations)


# speedup vs baseline: 1.1534x; 1.1534x over previous
import math

import jax
import jax.numpy as jnp
from jax import lax
from jax.experimental import pallas as pl
from jax.experimental.pallas import tpu as pltpu
from jax.experimental.pallas import tpu_sc as plsc

VOCAB = 1000000
NCAT = 16
B = 4096
L = 200
K = 50

NC = 2
NS = 16
NW = NC * NS
UPW = B // NW
GROUPS = UPW // 16

IW = 128
H_IDX = UPW * L
R_IDX = UPW * K
H_STREAMS = H_IDX // IW
R_STREAMS = R_IDX // IW
H_STREAMS_PER_GROUP = 16 * L // IW
R_STREAMS_PER_GROUP = 16 * K // IW

PACKED_N = VOCAB
ROWS_A = 31256
CHUNK_A = 768
NCH_A = -(-ROWS_A // CHUNK_A)
G16_A = CHUNK_A // 16


def _pack_body_sc(t128, packed_out, stage_v, pk_v):
    c = lax.axis_index("c")
    s = lax.axis_index("s")
    wid = c * NS + s

    lane = jax.lax.iota(jnp.int32, 16)
    start = wid * (VOCAB // NW)
    eff = (start // 8) * 8

    def ch_body(ch, _):
        base = jnp.minimum(ch * CHUNK_A, ROWS_A - CHUNK_A)
        pltpu.sync_copy(t128.at[pl.ds((eff + base) // 8, CHUNK_A // 8)], stage_v)

        def g_body(g, _):
            row = g * 16 + lane
            o = jnp.zeros((16,), jnp.int32)
            for cc in range(NCAT):
                flat = row * NCAT + cc
                col = plsc.load_gather(stage_v, [flat >> 7, flat & 127])
                o = o | (col << cc)
            pk_v[pl.ds(g * 16, 16)] = o
            return 0

        lax.fori_loop(0, G16_A, g_body, 0)
        pltpu.sync_copy(pk_v.at[pl.ds(0, CHUNK_A)],
                        packed_out.at[pl.ds(eff + base, CHUNK_A)])
        return 0

    lax.fori_loop(0, NCH_A, ch_body, 0)


_pack = pl.kernel(
    _pack_body_sc,
    out_type=jax.ShapeDtypeStruct((PACKED_N,), jnp.int32),
    mesh=plsc.VectorSubcoreMesh(
        core_axis_name="c", subcore_axis_name="s", num_cores=NC, num_subcores=NS
    ),
    scratch_types=[
        pltpu.VMEM((CHUNK_A // 8, 128), jnp.int32),
        pltpu.VMEM((CHUNK_A,), jnp.int32),
    ],
    compiler_params=pltpu.CompilerParams(
        use_tc_tiling_on_sc=False, needs_layout_passes=False
    ),
)


def _sc_body(packed, us_f, rec_f, p_out, q_out, idx_h, idx_r, words_h, words_r,
             psum_v, qsum_v, sem):
    c = lax.axis_index("c")
    s = lax.axis_index("s")
    wid = c * NS + s

    zero = jnp.zeros((16,), jnp.int32)
    lane = jax.lax.iota(jnp.int32, 16)

    pltpu.sync_copy(us_f.at[pl.ds(wid * H_IDX, H_IDX)], idx_h)
    pltpu.sync_copy(rec_f.at[pl.ds(wid * R_IDX, R_IDX)], idx_r)

    def fire(idx_ref, words_ref, chunk0, n):
        cps = [
            pltpu.make_async_copy(
                packed.at[idx_ref.at[pl.ds((chunk0 + j) * IW, IW)]],
                words_ref.at[pl.ds((chunk0 + j) * IW, IW)],
                sem,
            )
            for j in range(n)
        ]
        for cp in cps:
            cp.start()
        for cp in cps:
            cp.wait()

    def h_step(g, _):
        fire(idx_h, words_h, g * H_STREAMS_PER_GROUP, H_STREAMS_PER_GROUP)
        base = lane * L + g * 16 * L

        def r_body(r, accs):
            w = plsc.load_gather(words_h, [base + r])
            return tuple(accs[cc] + ((w >> cc) & 0x101) for cc in range(8))

        accs = lax.fori_loop(0, L, r_body, (zero,) * 8)
        out_base = (lane + g * 16) * NCAT
        for cc in range(8):
            plsc.store_scatter(psum_v, [out_base + cc], accs[cc] & 0xFF)
            plsc.store_scatter(psum_v, [out_base + cc + 8], (accs[cc] >> 8) & 0xFF)
        return 0

    lax.fori_loop(0, GROUPS, h_step, 0)

    def r_fire(st, _):
        fire(idx_r, words_r, st * 25, 25)
        return 0

    lax.fori_loop(0, R_STREAMS // 25, r_fire, 0)

    def r_step(g, _):
        base = lane * K + g * 16 * K

        def r_body(r, accs):
            w = plsc.load_gather(words_r, [base + r])
            return tuple(accs[cc] + ((w >> cc) & 0x101) for cc in range(8))

        accs = lax.fori_loop(0, K, r_body, (zero,) * 8)
        out_base = (lane + g * 16) * NCAT
        for cc in range(8):
            plsc.store_scatter(qsum_v, [out_base + cc], accs[cc] & 0xFF)
            plsc.store_scatter(qsum_v, [out_base + cc + 8], (accs[cc] >> 8) & 0xFF)
        return 0

    lax.fori_loop(0, GROUPS, r_step, 0)

    pltpu.sync_copy(psum_v, p_out.at[pl.ds(wid * UPW * NCAT, UPW * NCAT)])
    pltpu.sync_copy(qsum_v, q_out.at[pl.ds(wid * UPW * NCAT, UPW * NCAT)])


_sc_sums = pl.kernel(
    _sc_body,
    out_type=(
        jax.ShapeDtypeStruct((B * NCAT,), jnp.int32),
        jax.ShapeDtypeStruct((B * NCAT,), jnp.int32),
    ),
    mesh=plsc.VectorSubcoreMesh(
        core_axis_name="c", subcore_axis_name="s", num_cores=NC, num_subcores=NS
    ),
    scratch_types=[
        pltpu.VMEM((H_IDX,), jnp.int32),
        pltpu.VMEM((R_IDX,), jnp.int32),
        pltpu.VMEM((H_IDX,), jnp.int32),
        pltpu.VMEM((R_IDX,), jnp.int32),
        pltpu.VMEM((UPW * NCAT,), jnp.int32),
        pltpu.VMEM((UPW * NCAT,), jnp.int32),
        pltpu.SemaphoreType.DMA,
    ],
    compiler_params=pltpu.CompilerParams(
        use_tc_tiling_on_sc=False, needs_layout_passes=False
    ),
)


def _hell_body(p_ref, q_ref, o_ref):
    p = p_ref[...].astype(jnp.float32) * (1.0 / L)
    q = q_ref[...].astype(jnp.float32) * (1.0 / K)
    d = jnp.sqrt(p) - jnp.sqrt(q)
    o_ref[...] = jnp.sum(d * d, axis=1, keepdims=True) * (1.0 / math.sqrt(2.0))


_hell = pl.pallas_call(
    _hell_body,
    grid=(8,),
    in_specs=[
        pl.BlockSpec((B // 8, NCAT), lambda i: (i, 0)),
        pl.BlockSpec((B // 8, NCAT), lambda i: (i, 0)),
    ],
    out_specs=pl.BlockSpec((B // 8, 1), lambda i: (i, 0)),
    out_shape=jax.ShapeDtypeStruct((B, 1), jnp.float32),
)


@jax.jit
def _impl(item_categories, user_sequence, recommendations):
    packed = _pack(item_categories.reshape(VOCAB // 8, 128))
    us_f = user_sequence.reshape(B * L)
    rec_f = recommendations.reshape(B * K)
    p_sum, q_sum = _sc_sums(packed, us_f, rec_f)
    return _hell(p_sum.reshape(B, NCAT), q_sum.reshape(B, NCAT)).reshape(B)


def kernel(item_categories, user_sequence, recommendations):
    return _impl(item_categories, user_sequence, recommendations)


# speedup vs baseline: 1.3855x; 1.2013x over previous
import math

import jax
import jax.numpy as jnp
from jax import lax
from jax.experimental import pallas as pl
from jax.experimental.pallas import tpu as pltpu
from jax.experimental.pallas import tpu_sc as plsc

VOCAB = 1000000
NCAT = 16
B = 4096
L = 200
K = 50

NC = 2
NS = 16
NW = NC * NS
UPW = B // NW

IW = 128
CHUNK = 3200
NSTR = CHUNK // IW
H_IDX = UPW * L
R_IDX = UPW * K
H_CHUNKS = H_IDX // CHUNK
R_CHUNKS = R_IDX // CHUNK


def _sc_body(table, us_f, rec_f, p_out, q_out,
             idx0, idx1, rows0, rows1, psum_v, qsum_v, sem0, sem1):
    c = lax.axis_index("c")
    s = lax.axis_index("s")
    wid = c * NS + s

    zero = jnp.zeros((16,), jnp.int32)
    idx_bufs = (idx0, idx1)
    row_bufs = (rows0, rows1)
    sems = (sem0, sem1)

    chunks = []
    for hc in range(H_CHUNKS):
        chunks.append(("h", hc * CHUNK, CHUNK // L, L, hc * (CHUNK // L)))
    for rc in range(R_CHUNKS):
        chunks.append(("r", rc * CHUNK, CHUNK // K, K, rc * (CHUNK // K)))

    def stage_and_fire(ci):
        kind, off, _, _, _ = chunks[ci]
        buf = idx_bufs[ci % 2]
        rows = row_bufs[ci % 2]
        sem = sems[ci % 2]
        if kind == "h":
            pltpu.sync_copy(us_f.at[pl.ds(wid * H_IDX + off, CHUNK)], buf)
        else:
            pltpu.sync_copy(rec_f.at[pl.ds(wid * R_IDX + off, CHUNK)], buf)
        cps = [
            pltpu.make_async_copy(
                table.at[buf.at[pl.ds(j * IW, IW)]],
                rows.at[pl.ds(j * IW, IW)],
                sem,
            )
            for j in range(NSTR)
        ]
        for cp in cps:
            cp.start()
        return cps

    def drain(cps):
        for cp in cps:
            cp.wait()

    def accumulate(ci):
        _, _, n_users, run, out_base = chunks[ci]
        rows = row_bufs[ci % 2]
        out_ref = psum_v if chunks[ci][0] == "h" else qsum_v

        def u_body(u, _):
            base = u * run

            def r_body(r, accs):
                a0, a1 = accs
                t = base + r * 2
                return (a0 + rows[t], a1 + rows[t + 1])

            a0, a1 = lax.fori_loop(0, run // 2, r_body, (zero, zero))
            out_ref[out_base + u] = a0 + a1
            return 0

        lax.fori_loop(0, n_users, u_body, 0)

    n_chunks = len(chunks)
    inflight = stage_and_fire(0)
    for ci in range(n_chunks):
        nxt = None
        if ci + 1 < n_chunks:
            nxt = stage_and_fire(ci + 1)
        drain(inflight)
        accumulate(ci)
        inflight = nxt

    pltpu.sync_copy(psum_v, p_out.at[pl.ds(wid * UPW, UPW)])
    pltpu.sync_copy(qsum_v, q_out.at[pl.ds(wid * UPW, UPW)])


_sc_sums = pl.kernel(
    _sc_body,
    out_type=(
        jax.ShapeDtypeStruct((B, NCAT), jnp.int32),
        jax.ShapeDtypeStruct((B, NCAT), jnp.int32),
    ),
    mesh=plsc.VectorSubcoreMesh(
        core_axis_name="c", subcore_axis_name="s", num_cores=NC, num_subcores=NS
    ),
    scratch_types=[
        pltpu.VMEM((CHUNK,), jnp.int32),
        pltpu.VMEM((CHUNK,), jnp.int32),
        pltpu.VMEM((CHUNK, NCAT), jnp.int32),
        pltpu.VMEM((CHUNK, NCAT), jnp.int32),
        pltpu.VMEM((UPW, NCAT), jnp.int32),
        pltpu.VMEM((UPW, NCAT), jnp.int32),
        pltpu.SemaphoreType.DMA,
        pltpu.SemaphoreType.DMA,
    ],
    compiler_params=pltpu.CompilerParams(
        use_tc_tiling_on_sc=False, needs_layout_passes=False
    ),
)


def _hell_body(p_ref, q_ref, o_ref):
    p = p_ref[...].astype(jnp.float32) * (1.0 / L)
    q = q_ref[...].astype(jnp.float32) * (1.0 / K)
    d = jnp.sqrt(p) - jnp.sqrt(q)
    o_ref[...] = jnp.sum(d * d, axis=1, keepdims=True) * (1.0 / math.sqrt(2.0))


_hell = pl.pallas_call(
    _hell_body,
    grid=(8,),
    in_specs=[
        pl.BlockSpec((B // 8, NCAT), lambda i: (i, 0)),
        pl.BlockSpec((B // 8, NCAT), lambda i: (i, 0)),
    ],
    out_specs=pl.BlockSpec((B // 8, 1), lambda i: (i, 0)),
    out_shape=jax.ShapeDtypeStruct((B, 1), jnp.float32),
)


@jax.jit
def _impl(item_categories, user_sequence, recommendations):
    us_f = user_sequence.reshape(B * L)
    rec_f = recommendations.reshape(B * K)
    p_sum, q_sum = _sc_sums(item_categories, us_f, rec_f)
    return _hell(p_sum, q_sum).reshape(B)


def kernel(item_categories, user_sequence, recommendations):
    return _impl(item_categories, user_sequence, recommendations)
